# baseline (device time: 28583 ns/iter reference)
import jax
import jax.numpy as jnp
from jax import lax
from jax.experimental import pallas as pl
from jax.experimental.pallas import tpu as pltpu

N_DEV = 32
N_CHUNKS = 8
N_SLOTS = 4


def kernel(x, w_mat, scale_x, scale_w):
    m_per, k = x.shape
    _, n = w_mat.shape
    n_per = n // N_DEV
    n_chunk = n // N_CHUNKS
    tgt_per_chunk = N_DEV // N_CHUNKS

    def body(x_ref, w_hbm, sx_ref, sw_ref, out_ref,
             wbuf, commT, recvT, wdma_sems, send_sems, recv_sems, credit_sems):
        my = lax.axis_index("i")

        barrier_sem = pltpu.get_barrier_semaphore()
        pl.semaphore_signal(barrier_sem, inc=1)
        pl.semaphore_wait(barrier_sem, 1)

        for p in range(N_DEV):
            @pl.when(p != my)
            def _():
                pl.semaphore_signal(
                    credit_sems.at[my], inc=1,
                    device_id=(p,), device_id_type=pl.DeviceIdType.MESH,
                )

        def chunk_of(m):
            return lax.rem(my + m, N_CHUNKS)

        def w_dma(m, slot):
            off = pl.multiple_of(chunk_of(m) * n_chunk, n_chunk)
            return pltpu.make_async_copy(
                w_hbm.at[:, pl.ds(off, n_chunk)],
                wbuf.at[slot],
                wdma_sems.at[slot],
            )

        for m in range(min(N_SLOTS - 1, N_CHUNKS)):
            w_dma(m, m % N_SLOTS).start()

        s = sx_ref[0] * sw_ref[0]
        xv = x_ref[...]
        sends = []
        for m in range(N_CHUNKS):
            slot = m % N_SLOTS
            w_dma(m, slot).wait()
            accT = lax.dot_general(
                wbuf[slot], xv,
                dimension_numbers=(((0,), (1,)), ((), ())),
                preferred_element_type=jnp.float32,
            )
            nxt = m + N_SLOTS - 1
            if nxt < N_CHUNKS:
                w_dma(nxt, nxt % N_SLOTS).start()
            yT = accT * s
            yT = yT * jax.nn.sigmoid(yT)
            yTb = yT.astype(jnp.bfloat16)
            c = chunk_of(m)
            for u in range(tgt_per_chunk):
                j = c * tgt_per_chunk + u
                commT[j] = yTb[u * n_per:(u + 1) * n_per, :]
                rdma = pltpu.make_async_remote_copy(
                    src_ref=commT.at[j],
                    dst_ref=recvT.at[my],
                    send_sem=send_sems.at[j],
                    recv_sem=recv_sems.at[my],
                    device_id=(j,),
                    device_id_type=pl.DeviceIdType.MESH,
                )

                @pl.when(j != my)
                def _():
                    pl.semaphore_wait(credit_sems.at[j], 1)
                    rdma.start()

                @pl.when(j == my)
                def _():
                    recvT[j] = commT[j]

                sends.append((j, rdma))

        for u in range(N_DEV):
            @pl.when(u != my)
            def _():
                recv = pltpu.make_async_remote_copy(
                    src_ref=commT.at[u],
                    dst_ref=recvT.at[u],
                    send_sem=send_sems.at[u],
                    recv_sem=recv_sems.at[u],
                    device_id=(u,),
                    device_id_type=pl.DeviceIdType.MESH,
                )
                recv.wait_recv()
            out_ref[u * m_per:(u + 1) * m_per, :] = jnp.swapaxes(
                recvT[u], 0, 1).astype(jnp.float32)

        for j, rdma in sends:
            @pl.when(j != my)
            def _():
                rdma.wait_send()

    out_shape = jax.ShapeDtypeStruct((N_DEV * m_per, n_per), jnp.float32)
    return pl.pallas_call(
        body,
        out_shape=out_shape,
        in_specs=[
            pl.BlockSpec(memory_space=pltpu.VMEM),
            pl.BlockSpec(memory_space=pltpu.MemorySpace.HBM),
            pl.BlockSpec(memory_space=pltpu.SMEM),
            pl.BlockSpec(memory_space=pltpu.SMEM),
        ],
        out_specs=pl.BlockSpec(memory_space=pltpu.VMEM),
        scratch_shapes=[
            pltpu.VMEM((N_SLOTS, k, n_chunk), jnp.float32),
            pltpu.VMEM((N_DEV, n_per, m_per), jnp.bfloat16),
            pltpu.VMEM((N_DEV, n_per, m_per), jnp.bfloat16),
            pltpu.SemaphoreType.DMA((N_SLOTS,)),
            pltpu.SemaphoreType.DMA((N_DEV,)),
            pltpu.SemaphoreType.DMA((N_DEV,)),
            pltpu.SemaphoreType.REGULAR((N_DEV,)),
        ],
        compiler_params=pltpu.CompilerParams(
            vmem_limit_bytes=100 * 1024 * 1024,
            collective_id=0,
        ),
    )(x, w_mat, scale_x, scale_w)


# device time: 22663 ns/iter; 1.2612x vs baseline; 1.2612x over previous
import jax
import jax.numpy as jnp
from jax import lax
from jax.experimental import pallas as pl
from jax.experimental.pallas import tpu as pltpu

N_DEV = 32
N_CHUNKS = 8
N_SLOTS = 4


def kernel(x, w_mat, scale_x, scale_w):
    m_per, k = x.shape
    _, n = w_mat.shape
    n_per = n // N_DEV
    n_chunk = n // N_CHUNKS
    tgt_per_chunk = N_DEV // N_CHUNKS

    def body(x_ref, w_hbm, sx_ref, sw_ref, out_ref,
             wbuf, commT, recvT, wdma_sems, send_sems, recv_sems, credit_sems):
        my = lax.axis_index("i")

        barrier_sem = pltpu.get_barrier_semaphore()
        pl.semaphore_signal(barrier_sem, inc=1)
        pl.semaphore_wait(barrier_sem, 1)

        for p in range(N_DEV):
            @pl.when(p != my)
            def _():
                pl.semaphore_signal(
                    credit_sems.at[my], inc=1,
                    device_id=(p,), device_id_type=pl.DeviceIdType.MESH,
                )

        def chunk_of(m):
            return m

        def w_dma(m, slot):
            off = pl.multiple_of(chunk_of(m) * n_chunk, n_chunk)
            return pltpu.make_async_copy(
                w_hbm.at[:, pl.ds(off, n_chunk)],
                wbuf.at[slot],
                wdma_sems.at[slot],
            )

        for m in range(min(N_SLOTS - 1, N_CHUNKS)):
            w_dma(m, m % N_SLOTS).start()

        s = sx_ref[0] * sw_ref[0]
        xv = x_ref[...]
        sends = []
        for m in range(N_CHUNKS):
            slot = m % N_SLOTS
            w_dma(m, slot).wait()
            accT = lax.dot_general(
                wbuf[slot], xv,
                dimension_numbers=(((0,), (1,)), ((), ())),
                preferred_element_type=jnp.float32,
            )
            nxt = m + N_SLOTS - 1
            if nxt < N_CHUNKS:
                w_dma(nxt, nxt % N_SLOTS).start()
            yT = accT * s
            yT = yT * jax.nn.sigmoid(yT)
            yTb = yT.astype(jnp.bfloat16)
            c = chunk_of(m)
            for u in range(tgt_per_chunk):
                j = c * tgt_per_chunk + u
                commT[j] = yTb[u * n_per:(u + 1) * n_per, :]
                rdma = pltpu.make_async_remote_copy(
                    src_ref=commT.at[j],
                    dst_ref=recvT.at[my],
                    send_sem=send_sems.at[j],
                    recv_sem=recv_sems.at[my],
                    device_id=(j,),
                    device_id_type=pl.DeviceIdType.MESH,
                )

                @pl.when(j != my)
                def _():
                    pl.semaphore_wait(credit_sems.at[j], 1)
                    rdma.start()

                @pl.when(j == my)
                def _():
                    recvT[j] = commT[j]

                sends.append((j, rdma))

        for u in range(N_DEV):
            @pl.when(u != my)
            def _():
                recv = pltpu.make_async_remote_copy(
                    src_ref=commT.at[u],
                    dst_ref=recvT.at[u],
                    send_sem=send_sems.at[u],
                    recv_sem=recv_sems.at[u],
                    device_id=(u,),
                    device_id_type=pl.DeviceIdType.MESH,
                )
                recv.wait_recv()
            out_ref[u * m_per:(u + 1) * m_per, :] = jnp.swapaxes(
                recvT[u], 0, 1).astype(jnp.float32)

        for j, rdma in sends:
            @pl.when(j != my)
            def _():
                rdma.wait_send()

    out_shape = jax.ShapeDtypeStruct((N_DEV * m_per, n_per), jnp.float32)
    return pl.pallas_call(
        body,
        out_shape=out_shape,
        in_specs=[
            pl.BlockSpec(memory_space=pltpu.VMEM),
            pl.BlockSpec(memory_space=pltpu.MemorySpace.HBM),
            pl.BlockSpec(memory_space=pltpu.SMEM),
            pl.BlockSpec(memory_space=pltpu.SMEM),
        ],
        out_specs=pl.BlockSpec(memory_space=pltpu.VMEM),
        scratch_shapes=[
            pltpu.VMEM((N_SLOTS, k, n_chunk), jnp.float32),
            pltpu.VMEM((N_DEV, n_per, m_per), jnp.bfloat16),
            pltpu.VMEM((N_DEV, n_per, m_per), jnp.bfloat16),
            pltpu.SemaphoreType.DMA((N_SLOTS,)),
            pltpu.SemaphoreType.DMA((N_DEV,)),
            pltpu.SemaphoreType.DMA((N_DEV,)),
            pltpu.SemaphoreType.REGULAR((N_DEV,)),
        ],
        compiler_params=pltpu.CompilerParams(
            vmem_limit_bytes=100 * 1024 * 1024,
            collective_id=0,
        ),
    )(x, w_mat, scale_x, scale_w)
